# baseline (speedup 1.0000x reference)
import jax
import jax.numpy as jnp
from jax.experimental import pallas as pl
from jax.experimental.pallas import tpu as pltpu

_FORWARD_WEIGHT = 1.0
_BACKWARD_WEIGHT = 1.0
_MB = 2048


def _chamfer_kernel(pred_ref, tgt_t_ref, loss_ref, fwd_scratch, lhs_scratch):
    m = pl.program_id(1)
    num_m = pl.num_programs(1)

    @pl.when(m == 0)
    def _():
        px = pred_ref[0, :, 0:1]
        py = pred_ref[0, :, 1:2]
        pz = pred_ref[0, :, 2:3]
        pn = px * px + py * py + pz * pz
        h = pn.astype(jnp.bfloat16)
        r1 = pn - h.astype(jnp.float32)
        mi = r1.astype(jnp.bfloat16)
        lo = (r1 - mi.astype(jnp.float32)).astype(jnp.bfloat16)
        pb = pred_ref[0, :, :].astype(jnp.bfloat16) * jnp.bfloat16(2.0)
        lhs_scratch[:, :] = jnp.concatenate([pb, -h, -mi, -lo], axis=1)

    tx = tgt_t_ref[0, 0:1, :]
    ty = tgt_t_ref[0, 1:2, :]
    tz = tgt_t_ref[0, 2:3, :]
    tn = tx * tx + ty * ty + tz * tz

    tb = tgt_t_ref[0, :, :].astype(jnp.bfloat16)
    ones = jnp.ones((3, tb.shape[1]), jnp.bfloat16)
    rhs = jnp.concatenate([tb, ones], axis=0)

    s = jax.lax.dot_general(lhs_scratch[:, :], rhs, (((1,), (0,)), ((), ())),
                            preferred_element_type=jnp.float32)

    col_min = tn - jnp.max(s, axis=0, keepdims=True)
    bwd_sum = jnp.sum(jnp.sqrt(jnp.maximum(col_min, 1e-12)), keepdims=True)

    row_min = jnp.min(tn - s, axis=1, keepdims=True)

    @pl.when(m == 0)
    def _():
        loss_ref[0, :, :] = jnp.zeros((1, 1), jnp.float32)
        fwd_scratch[:, :] = row_min

    @pl.when(m > 0)
    def _():
        fwd_scratch[:, :] = jnp.minimum(fwd_scratch[:, :], row_min)

    loss_ref[0, :, :] += _BACKWARD_WEIGHT * bwd_sum

    @pl.when(m == num_m - 1)
    def _():
        fwd = jnp.sqrt(jnp.maximum(fwd_scratch[:, :], 1e-12))
        loss_ref[0, :, :] += _FORWARD_WEIGHT * jnp.sum(fwd, keepdims=True)


def kernel(pred, target):
    if pred.ndim == 2:
        pred = pred[None, ...]
    if target.ndim == 2:
        target = target[None, ...]
    B, N, D = pred.shape
    _, M, _ = target.shape
    tgt_t = jnp.swapaxes(target, 1, 2)
    mb = min(_MB, M)
    grid = (B, M // mb)
    loss = pl.pallas_call(
        _chamfer_kernel,
        grid=grid,
        in_specs=[
            pl.BlockSpec((1, N, D), lambda b, m: (b, 0, 0)),
            pl.BlockSpec((1, D, mb), lambda b, m: (b, 0, m)),
        ],
        out_specs=pl.BlockSpec((1, 1, 1), lambda b, m: (b, 0, 0)),
        out_shape=jax.ShapeDtypeStruct((B, 1, 1), jnp.float32),
        scratch_shapes=[
            pltpu.VMEM((N, 1), jnp.float32),
            pltpu.VMEM((N, 6), jnp.bfloat16),
        ],
    )(pred, tgt_t)
    return jnp.sum(loss) / B


# speedup vs baseline: 1.1032x; 1.1032x over previous
import jax
import jax.numpy as jnp
from jax.experimental import pallas as pl
from jax.experimental.pallas import tpu as pltpu

_FORWARD_WEIGHT = 1.0
_BACKWARD_WEIGHT = 1.0
_MB = 4096


def _chamfer_kernel(pred_ref, tgt_t_ref, loss_ref, fwd_scratch, pn_scratch):
    m = pl.program_id(1)
    num_m = pl.num_programs(1)

    @pl.when(m == 0)
    def _():
        px = pred_ref[0, :, 0:1]
        py = pred_ref[0, :, 1:2]
        pz = pred_ref[0, :, 2:3]
        pn_scratch[:, :] = px * px + py * py + pz * pz

    tx = tgt_t_ref[0, 0:1, :]
    ty = tgt_t_ref[0, 1:2, :]
    tz = tgt_t_ref[0, 2:3, :]
    tn = tx * tx + ty * ty + tz * tz

    pb = pred_ref[0, :, :].astype(jnp.bfloat16) * jnp.bfloat16(2.0)
    tb = tgt_t_ref[0, :, :].astype(jnp.bfloat16)
    pt2 = jax.lax.dot_general(pb, tb, (((1,), (0,)), ((), ())),
                              preferred_element_type=jnp.float32)

    pn = pn_scratch[:, :]

    col_min = jnp.min(pn - pt2, axis=0, keepdims=True) + tn
    bwd_sum = jnp.sum(jnp.sqrt(jnp.maximum(col_min, 1e-12)), keepdims=True)

    row_min = jnp.min(tn - pt2, axis=1, keepdims=True)

    @pl.when(m == 0)
    def _():
        loss_ref[0, :, :] = jnp.zeros((1, 1), jnp.float32)
        fwd_scratch[:, :] = row_min

    @pl.when(m > 0)
    def _():
        fwd_scratch[:, :] = jnp.minimum(fwd_scratch[:, :], row_min)

    loss_ref[0, :, :] += _BACKWARD_WEIGHT * bwd_sum

    @pl.when(m == num_m - 1)
    def _():
        fwd = jnp.sqrt(jnp.maximum(fwd_scratch[:, :] + pn, 1e-12))
        loss_ref[0, :, :] += _FORWARD_WEIGHT * jnp.sum(fwd, keepdims=True)


def kernel(pred, target):
    if pred.ndim == 2:
        pred = pred[None, ...]
    if target.ndim == 2:
        target = target[None, ...]
    B, N, D = pred.shape
    _, M, _ = target.shape
    tgt_t = jnp.swapaxes(target, 1, 2)
    mb = min(_MB, M)
    grid = (B, M // mb)
    loss = pl.pallas_call(
        _chamfer_kernel,
        grid=grid,
        in_specs=[
            pl.BlockSpec((1, N, D), lambda b, m: (b, 0, 0)),
            pl.BlockSpec((1, D, mb), lambda b, m: (b, 0, m)),
        ],
        out_specs=pl.BlockSpec((1, 1, 1), lambda b, m: (b, 0, 0)),
        out_shape=jax.ShapeDtypeStruct((B, 1, 1), jnp.float32),
        scratch_shapes=[
            pltpu.VMEM((N, 1), jnp.float32),
            pltpu.VMEM((N, 1), jnp.float32),
        ],
        compiler_params=pltpu.CompilerParams(
            vmem_limit_bytes=100 * 1024 * 1024),
    )(pred, tgt_t)
    return jnp.sum(loss) / B
